# baseline (speedup 1.0000x reference)
import jax
import jax.numpy as jnp
from jax import lax
from jax.experimental import pallas as pl
from jax.experimental.pallas import tpu as pltpu
from jax.experimental.pallas import tpu_sc as plsc

_B, _H, _W = 8, 512, 512
_N = _B * _H * _W
_NEG_RATIO = 3.0
_EPS = 1e-06


_NCORES, _NSUB = 2, 16
_NW = _NCORES * _NSUB
_PER = _N // _NW
_CHUNK = 8192
_NCH = _PER // _CHUNK
_VPG = 8
_LANES = 16

_LOG1P_C = (-1.7414116888e-02, 8.2691420711e-02, -1.9035463582e-01,
            3.1574753796e-01, -4.9737329285e-01, 9.9984770861e-01,
            1.4716138946e-06)


def _sc_body(x_hbm, gt_hbm, m_hbm, stats_out, neg_out,
             xb, gb, mb, ob, sbuf, semi, semo):
    c = lax.axis_index("c")
    s = lax.axis_index("s")
    w = c * _NSUB + s
    base_w = w * _PER

    def _start_in(ch):
        b = ch % 2
        base = base_w + ch * _CHUNK
        return (pltpu.async_copy(x_hbm.at[pl.ds(base, _CHUNK)],
                                 xb.at[b], semi.at[b, 0]),
                pltpu.async_copy(gt_hbm.at[pl.ds(base, _CHUNK)],
                                 gb.at[b], semi.at[b, 1]),
                pltpu.async_copy(m_hbm.at[pl.ds(base, _CHUNK)],
                                 mb.at[b], semi.at[b, 2]))

    zero = jnp.zeros((_LANES,), jnp.float32)
    accs = (zero, zero, zero, zero)

    pending = _start_in(0)
    out_cp = [None, None]
    for ch in range(_NCH):
        b = ch % 2
        for cp in pending:
            cp.wait()
        if ch + 1 < _NCH:
            pending = _start_in(ch + 1)

        @plsc.parallel_loop(0, _CHUNK // _LANES, step=_VPG, carry=accs)
        def _step(i, accs):
            a, bb, cc, d = accs
            ta = tb = tc = td = None
            for j in range(_VPG):
                off = (i + j) * _LANES
                xv = xb[b, pl.ds(off, _LANES)]
                gi = gb[b, pl.ds(off, _LANES)]
                mi = mb[b, pl.ds(off, _LANES)]
                gtv = jnp.where(gi > 0, 1.0, 0.0)
                mv = mi.astype(jnp.float32)
                e = jnp.exp(-jnp.abs(xv))
                p = jnp.full((_LANES,), _LOG1P_C[0], jnp.float32)
                for coef in _LOG1P_C[1:]:
                    p = p * e + jnp.float32(coef)
                sp = jnp.maximum(xv, 0.0) + p
                gm = gtv * mv
                lossv = sp - xv * gtv
                negv = sp * (mv - gm)
                ta = gm if ta is None else ta + gm
                tb = mv if tb is None else tb + mv
                tc = lossv * gm if tc is None else tc + lossv * gm
                td = negv if td is None else td + negv
                ob[b, pl.ds(off, _LANES)] = negv
            return (a + ta, bb + tb, cc + tc, d + td)

        accs = _step
        if out_cp[b] is not None:
            out_cp[b].wait()
        base = base_w + ch * _CHUNK
        out_cp[b] = pltpu.async_copy(
            ob.at[b], neg_out.at[pl.ds(base, _CHUNK)], semo.at[b])
    for cp in out_cp:
        if cp is not None:
            cp.wait()

    sbuf[0, :] = accs[0]
    sbuf[1, :] = accs[1]
    sbuf[2, :] = accs[2]
    sbuf[3, :] = accs[3]
    pltpu.sync_copy(sbuf, stats_out.at[w])


def _sc_pass(xf, gtf, mf):
    mesh = plsc.VectorSubcoreMesh(core_axis_name="c", subcore_axis_name="s")
    return pl.kernel(
        _sc_body,
        out_type=[jax.ShapeDtypeStruct((_NW, 4, _LANES), jnp.float32),
                  jax.ShapeDtypeStruct((_N,), jnp.float32)],
        mesh=mesh,
        scratch_types=[
            pltpu.VMEM((2, _CHUNK), jnp.float32),
            pltpu.VMEM((2, _CHUNK), jnp.int32),
            pltpu.VMEM((2, _CHUNK), jnp.int32),
            pltpu.VMEM((2, _CHUNK), jnp.float32),
            pltpu.VMEM((4, _LANES), jnp.float32),
            pltpu.SemaphoreType.DMA((2, 3)),
            pltpu.SemaphoreType.DMA((2,)),
        ],
    )(xf, gtf, mf)


_ROWS, _COLS = 2048, 1024
_TILE = 256
_NT = _ROWS // _TILE
_NROUNDS = 16

_S_C1, _S_C2, _S_C3, _S_RELU, _S_KF = range(5)


def _sel_body(k_ref, neg_ref, out_ref, smf, smi):
    r = pl.program_id(0)
    t = pl.program_id(1)

    @pl.when((r == 0) & (t == 0))
    def _init():
        smi[0] = 0
        smf[_S_KF] = k_ref[0].astype(jnp.float32)
        smf[_S_RELU] = 0.0

    @pl.when(r < _NROUNDS)
    def _bisect():
        step = jnp.maximum(jnp.int32(1), jnp.int32(1 << 29) >> (2 * r))

        @pl.when(t == 0)
        def _zero_counts():
            smf[_S_C1] = 0.0
            smf[_S_C2] = 0.0
            smf[_S_C3] = 0.0

        lo = smi[0]
        bits = jax.lax.bitcast_convert_type(neg_ref[...], jnp.int32)
        smf[_S_C1] += jnp.sum((bits >= lo + step).astype(jnp.float32))
        smf[_S_C2] += jnp.sum((bits >= lo + 2 * step).astype(jnp.float32))
        smf[_S_C3] += jnp.sum((bits >= lo + 3 * step).astype(jnp.float32))

        @pl.when(t == _NT - 1)
        def _decide():
            kf = smf[_S_KF]
            jmax = ((smf[_S_C1] >= kf).astype(jnp.int32)
                    + (smf[_S_C2] >= kf).astype(jnp.int32)
                    + (smf[_S_C3] >= kf).astype(jnp.int32))
            smi[0] = lo + jmax * step

    @pl.when(r == _NROUNDS)
    def _final():
        tstar = jax.lax.bitcast_convert_type(smi[0], jnp.float32)
        smf[_S_RELU] += jnp.sum(jnp.maximum(neg_ref[...] - tstar, 0.0))

        @pl.when(t == _NT - 1)
        def _assemble():
            kf = smf[_S_KF]
            out_ref[0] = jnp.where(
                k_ref[0] > 0, smf[_S_RELU] + kf * tstar, 0.0)


def _tc_topk_sum(neg2d, k):
    return pl.pallas_call(
        _sel_body,
        grid=(_NROUNDS + 1, _NT),
        in_specs=[
            pl.BlockSpec(memory_space=pltpu.SMEM),
            pl.BlockSpec((_TILE, _COLS), lambda r, t: (t, 0)),
        ],
        out_specs=pl.BlockSpec(memory_space=pltpu.SMEM),
        out_shape=jax.ShapeDtypeStruct((1,), jnp.float32),
        scratch_shapes=[
            pltpu.SMEM((5,), jnp.float32),
            pltpu.SMEM((1,), jnp.int32),
        ],
        compiler_params=pltpu.CompilerParams(
            dimension_semantics=("arbitrary", "arbitrary")),
    )(k.reshape(1), neg2d)[0]


def kernel(preds, downsample_ratio, gt_shrink, gt_shrink_mask):
    xf = preds.reshape(_N)
    gtf = gt_shrink.reshape(_N)
    mf = gt_shrink_mask.reshape(_N)
    stats, neg = _sc_pass(xf, gtf, mf)

    st = jnp.sum(stats, axis=(0, 2))
    pos_f, mask_f, pos_loss, neg_sum = st[0], st[1], st[2], st[3]
    neg_i = (mask_f - pos_f).astype(jnp.int32)
    cap = (pos_f * _NEG_RATIO).astype(jnp.int32)
    k = jnp.minimum(neg_i, cap)

    neg_top = lax.cond(
        cap < neg_i,
        lambda: _tc_topk_sum(neg.reshape(_ROWS, _COLS), k),
        lambda: neg_sum)

    denom = (pos_f.astype(jnp.int32) + k).astype(jnp.float32) + _EPS
    loss = (pos_loss + neg_top) / denom
    return loss * downsample_ratio


# speedup vs baseline: 1.8948x; 1.8948x over previous
import jax
import jax.numpy as jnp
from jax import lax
from jax.experimental import pallas as pl
from jax.experimental.pallas import tpu as pltpu
from jax.experimental.pallas import tpu_sc as plsc

_B, _H, _W = 8, 512, 512
_N = _B * _H * _W
_ROWS, _COLS = 2048, 1024
_NEG_RATIO = 3.0
_EPS = 1e-06


_NCORES, _NSUB = 2, 16
_NW = _NCORES * _NSUB
_SC_ROWS = 512
_SC_N = _SC_ROWS * _COLS
_CHUNK = _SC_N // _NW
_VPG = 8
_LANES = 16

_LOG1P_C = (-1.7414116888e-02, 8.2691420711e-02, -1.9035463582e-01,
            3.1574753796e-01, -4.9737329285e-01, 9.9984770861e-01,
            1.4716138946e-06)


def _softplus_parts(xv, gi, mi):
    gtv = jnp.where(gi > 0, 1.0, 0.0)
    mv = mi.astype(jnp.float32)
    e = jnp.exp(-jnp.abs(xv))
    p = jnp.full((_LANES,), _LOG1P_C[0], jnp.float32)
    for coef in _LOG1P_C[1:]:
        p = p * e + jnp.float32(coef)
    sp = jnp.maximum(xv, 0.0) + p
    gm = gtv * mv
    lossv = sp - xv * gtv
    negv = sp * (mv - gm)
    return gm, mv, lossv * gm, negv


def _sc_body(x_hbm, gt_hbm, m_hbm, stats_out, xb, gb, mb, sbuf,
             semx, semg, semm):
    c = lax.axis_index("c")
    s = lax.axis_index("s")
    w = c * _NSUB + s
    base = w * _CHUNK

    cpx = pltpu.async_copy(x_hbm.at[pl.ds(base, _CHUNK)], xb, semx)
    cpg = pltpu.async_copy(gt_hbm.at[pl.ds(base, _CHUNK)], gb, semg)
    cpm = pltpu.async_copy(m_hbm.at[pl.ds(base, _CHUNK)], mb, semm)
    cpx.wait()
    cpg.wait()
    cpm.wait()

    zero = jnp.zeros((_LANES,), jnp.float32)

    def _step(i, accs):
        a, b, cc, d = accs
        for j in range(_VPG):
            off = (i * _VPG + j) * _LANES
            gm, mv, plv, negv = _softplus_parts(
                xb[pl.ds(off, _LANES)],
                gb[pl.ds(off, _LANES)],
                mb[pl.ds(off, _LANES)])
            a = a + gm
            b = b + mv
            cc = cc + plv
            d = d + negv
        return (a, b, cc, d)

    accs = lax.fori_loop(0, _CHUNK // (_VPG * _LANES), _step,
                         (zero, zero, zero, zero))
    sbuf[0, :] = accs[0]
    sbuf[1, :] = accs[1]
    sbuf[2, :] = accs[2]
    sbuf[3, :] = accs[3]
    pltpu.sync_copy(sbuf, stats_out.at[w])


def _sc_stats(xf, gtf, mf):
    mesh = plsc.VectorSubcoreMesh(core_axis_name="c", subcore_axis_name="s")
    return pl.kernel(
        _sc_body,
        out_type=jax.ShapeDtypeStruct((_NW, 4, _LANES), jnp.float32),
        mesh=mesh,
        scratch_types=[
            pltpu.VMEM((_CHUNK,), jnp.float32),
            pltpu.VMEM((_CHUNK,), jnp.int32),
            pltpu.VMEM((_CHUNK,), jnp.int32),
            pltpu.VMEM((4, _LANES), jnp.float32),
            pltpu.SemaphoreType.DMA,
            pltpu.SemaphoreType.DMA,
            pltpu.SemaphoreType.DMA,
        ],
    )(xf, gtf, mf)


_TILE = 256
_TC_NT = (_ROWS - _SC_ROWS) // _TILE


def _tc_stats_body(x_ref, gt_ref, m_ref, out_ref, smf):
    t = pl.program_id(0)

    @pl.when(t == 0)
    def _init():
        smf[0] = 0.0
        smf[1] = 0.0
        smf[2] = 0.0
        smf[3] = 0.0

    x = x_ref[...]
    gt = (gt_ref[...] > 0).astype(jnp.float32)
    m = m_ref[...].astype(jnp.float32)
    loss = jnp.maximum(x, 0.0) - x * gt + jnp.log1p(jnp.exp(-jnp.abs(x)))
    pos = gt * m
    neg_loss = loss * (m - pos)
    smf[0] += jnp.sum(pos)
    smf[1] += jnp.sum(m)
    smf[2] += jnp.sum(loss * pos)
    smf[3] += jnp.sum(neg_loss)

    @pl.when(t == _TC_NT - 1)
    def _fin():
        out_ref[0] = smf[0]
        out_ref[1] = smf[1]
        out_ref[2] = smf[2]
        out_ref[3] = smf[3]


def _tc_stats(x2d, gt2d, m2d):
    off = _SC_ROWS // _TILE
    return pl.pallas_call(
        _tc_stats_body,
        grid=(_TC_NT,),
        in_specs=[
            pl.BlockSpec((_TILE, _COLS), lambda t: (t + off, 0)),
            pl.BlockSpec((_TILE, _COLS), lambda t: (t + off, 0)),
            pl.BlockSpec((_TILE, _COLS), lambda t: (t + off, 0)),
        ],
        out_specs=pl.BlockSpec(memory_space=pltpu.SMEM),
        out_shape=jax.ShapeDtypeStruct((4,), jnp.float32),
        scratch_shapes=[pltpu.SMEM((4,), jnp.float32)],
        compiler_params=pltpu.CompilerParams(
            dimension_semantics=("arbitrary",)),
    )(x2d, gt2d, m2d)


_NROUNDS = 16

(_S_POS_CNT, _S_MASK_CNT, _S_POS_LOSS, _S_NEG_LOSS, _S_C1, _S_C2, _S_C3,
 _S_RELU, _S_KF) = range(9)
_I_LO, _I_K, _I_SLOW = range(3)
_FB_NT = _ROWS // _TILE


def _fb_body(x_ref, gt_ref, m_ref, out_ref, slab, smf, smi):
    r = pl.program_id(0)
    t = pl.program_id(1)

    @pl.when(r == 0)
    def _pass1():
        @pl.when(t == 0)
        def _init():
            smf[_S_POS_CNT] = 0.0
            smf[_S_MASK_CNT] = 0.0
            smf[_S_POS_LOSS] = 0.0

        x = x_ref[...]
        gt = (gt_ref[...] > 0).astype(jnp.float32)
        m = m_ref[...].astype(jnp.float32)
        loss = jnp.maximum(x, 0.0) - x * gt + jnp.log1p(jnp.exp(-jnp.abs(x)))
        pos = gt * m
        neg_loss = loss * (m - pos)
        smf[_S_POS_CNT] += jnp.sum(pos)
        smf[_S_MASK_CNT] += jnp.sum(m)
        smf[_S_POS_LOSS] += jnp.sum(loss * pos)
        slab[pl.ds(t * _TILE, _TILE), :] = neg_loss

    @pl.when((r == 1) & (t == 0))
    def _init_k():
        pos_f = smf[_S_POS_CNT]
        neg_i = (smf[_S_MASK_CNT] - pos_f).astype(jnp.int32)
        cap = (pos_f * _NEG_RATIO).astype(jnp.int32)
        k = jnp.minimum(neg_i, cap)
        smi[_I_K] = k
        smf[_S_KF] = k.astype(jnp.float32)
        smi[_I_LO] = 0
        smf[_S_RELU] = 0.0

    @pl.when((r >= 1) & (r <= _NROUNDS))
    def _bisect():
        i = r - 1
        step = jnp.maximum(jnp.int32(1), jnp.int32(1 << 29) >> (2 * i))

        @pl.when(t == 0)
        def _zero_counts():
            smf[_S_C1] = 0.0
            smf[_S_C2] = 0.0
            smf[_S_C3] = 0.0

        lo = smi[_I_LO]
        bits = jax.lax.bitcast_convert_type(
            slab[pl.ds(t * _TILE, _TILE), :], jnp.int32)
        smf[_S_C1] += jnp.sum((bits >= lo + step).astype(jnp.float32))
        smf[_S_C2] += jnp.sum((bits >= lo + 2 * step).astype(jnp.float32))
        smf[_S_C3] += jnp.sum((bits >= lo + 3 * step).astype(jnp.float32))

        @pl.when(t == _FB_NT - 1)
        def _decide():
            kf = smf[_S_KF]
            jmax = ((smf[_S_C1] >= kf).astype(jnp.int32)
                    + (smf[_S_C2] >= kf).astype(jnp.int32)
                    + (smf[_S_C3] >= kf).astype(jnp.int32))
            smi[_I_LO] = lo + jmax * step

    @pl.when(r == _NROUNDS + 1)
    def _final():
        tstar = jax.lax.bitcast_convert_type(smi[_I_LO], jnp.float32)
        v = slab[pl.ds(t * _TILE, _TILE), :]
        smf[_S_RELU] += jnp.sum(jnp.maximum(v - tstar, 0.0))

        @pl.when(t == _FB_NT - 1)
        def _assemble():
            k = smi[_I_K]
            kf = smf[_S_KF]
            neg_top = jnp.where(k > 0, smf[_S_RELU] + kf * tstar, 0.0)
            pos_i = smf[_S_POS_CNT].astype(jnp.int32)
            denom = (pos_i + k).astype(jnp.float32) + _EPS
            out_ref[0] = (smf[_S_POS_LOSS] + neg_top) / denom


def _fallback_full(x2d, gt2d, m2d):
    return pl.pallas_call(
        _fb_body,
        grid=(_NROUNDS + 2, _FB_NT),
        in_specs=[
            pl.BlockSpec((_TILE, _COLS),
                         lambda r, t: (jnp.where(r == 0, t, 0), 0)),
            pl.BlockSpec((_TILE, _COLS),
                         lambda r, t: (jnp.where(r == 0, t, 0), 0)),
            pl.BlockSpec((_TILE, _COLS),
                         lambda r, t: (jnp.where(r == 0, t, 0), 0)),
        ],
        out_specs=pl.BlockSpec(memory_space=pltpu.SMEM),
        out_shape=jax.ShapeDtypeStruct((1,), jnp.float32),
        scratch_shapes=[
            pltpu.VMEM((_ROWS, _COLS), jnp.float32),
            pltpu.SMEM((9,), jnp.float32),
            pltpu.SMEM((3,), jnp.int32),
        ],
        compiler_params=pltpu.CompilerParams(
            dimension_semantics=("arbitrary", "arbitrary")),
    )(x2d, gt2d, m2d)[0]


def kernel(preds, downsample_ratio, gt_shrink, gt_shrink_mask):
    x2d = preds.reshape(_ROWS, _COLS)
    gt2d = gt_shrink.reshape(_ROWS, _COLS)
    m2d = gt_shrink_mask.reshape(_ROWS, _COLS)

    sc_stats = _sc_stats(preds.reshape(_N), gt_shrink.reshape(_N),
                         gt_shrink_mask.reshape(_N))
    tc_stats = _tc_stats(x2d, gt2d, m2d)

    st = jnp.sum(sc_stats, axis=(0, 2)) + tc_stats
    pos_f, mask_f, pos_loss, neg_sum = st[0], st[1], st[2], st[3]
    neg_i = (mask_f - pos_f).astype(jnp.int32)
    cap = (pos_f * _NEG_RATIO).astype(jnp.int32)
    k = jnp.minimum(neg_i, cap)

    denom = (pos_f.astype(jnp.int32) + k).astype(jnp.float32) + _EPS
    fast_loss = (pos_loss + neg_sum) / denom

    loss = lax.cond(
        cap < neg_i,
        lambda: _fallback_full(x2d, gt2d, m2d),
        lambda: fast_loss)
    return loss * downsample_ratio


# speedup vs baseline: 3.4959x; 1.8450x over previous
import jax
import jax.numpy as jnp
from jax.experimental import pallas as pl
from jax.experimental.pallas import tpu as pltpu

_B, _H, _W = 8, 512, 512
_N = _B * _H * _W
_ROWS, _COLS = 2048, 1024
_TILE = 256
_NT = _ROWS // _TILE
_NROUNDS = 16
_NEG_RATIO = 3.0
_EPS = 1e-06

(_S_POS_CNT, _S_MASK_CNT, _S_POS_LOSS, _S_NEG_LOSS, _S_C1, _S_C2, _S_C3,
 _S_RELU, _S_KF) = range(9)
_I_LO, _I_K, _I_SLOW = range(3)


def _bce_body(x_ref, gt_ref, m_ref, out_ref, slab, smf, smi):
    r = pl.program_id(0)
    t = pl.program_id(1)

    @pl.when(r == 0)
    def _pass1():
        @pl.when(t == 0)
        def _init():
            smf[_S_POS_CNT] = 0.0
            smf[_S_MASK_CNT] = 0.0
            smf[_S_POS_LOSS] = 0.0
            smf[_S_NEG_LOSS] = 0.0

        x = x_ref[...]
        gt = (gt_ref[...].astype(jnp.float32) > 0.0).astype(jnp.float32)
        m = m_ref[...].astype(jnp.float32)
        loss = jnp.maximum(x, 0.0) - x * gt + jnp.log1p(jnp.exp(-jnp.abs(x)))
        pos = gt * m
        neg_loss = loss * (m - pos)
        smf[_S_POS_CNT] += jnp.sum(pos)
        smf[_S_MASK_CNT] += jnp.sum(m)
        smf[_S_POS_LOSS] += jnp.sum(loss * pos)
        smf[_S_NEG_LOSS] += jnp.sum(neg_loss)
        slab[pl.ds(t * _TILE, _TILE), :] = neg_loss

    @pl.when((r == 1) & (t == 0))
    def _init_k():
        pos_f = smf[_S_POS_CNT]
        neg_i = (smf[_S_MASK_CNT] - pos_f).astype(jnp.int32)
        cap = (pos_f * _NEG_RATIO).astype(jnp.int32)
        k = jnp.minimum(neg_i, cap)
        smi[_I_K] = k
        smf[_S_KF] = k.astype(jnp.float32)
        smi[_I_LO] = 0
        smi[_I_SLOW] = (cap < neg_i).astype(jnp.int32)
        smf[_S_RELU] = 0.0

    @pl.when((r >= 1) & (r <= _NROUNDS))
    def _bisect():
        @pl.when(smi[_I_SLOW] == 1)
        def _do_round():
            i = r - 1
            step = jnp.maximum(jnp.int32(1), jnp.int32(1 << 29) >> (2 * i))

            @pl.when(t == 0)
            def _zero_counts():
                smf[_S_C1] = 0.0
                smf[_S_C2] = 0.0
                smf[_S_C3] = 0.0

            lo = smi[_I_LO]
            bits = jax.lax.bitcast_convert_type(
                slab[pl.ds(t * _TILE, _TILE), :], jnp.int32)
            smf[_S_C1] += jnp.sum((bits >= lo + step).astype(jnp.float32))
            smf[_S_C2] += jnp.sum((bits >= lo + 2 * step).astype(jnp.float32))
            smf[_S_C3] += jnp.sum((bits >= lo + 3 * step).astype(jnp.float32))

            @pl.when(t == _NT - 1)
            def _decide():
                kf = smf[_S_KF]
                jmax = ((smf[_S_C1] >= kf).astype(jnp.int32)
                        + (smf[_S_C2] >= kf).astype(jnp.int32)
                        + (smf[_S_C3] >= kf).astype(jnp.int32))
                smi[_I_LO] = lo + jmax * step

    @pl.when(r == _NROUNDS + 1)
    def _final():
        tstar = jax.lax.bitcast_convert_type(smi[_I_LO], jnp.float32)

        @pl.when(smi[_I_SLOW] == 1)
        def _relu_pass():
            v = slab[pl.ds(t * _TILE, _TILE), :]
            smf[_S_RELU] += jnp.sum(jnp.maximum(v - tstar, 0.0))

        @pl.when(t == _NT - 1)
        def _assemble():
            k = smi[_I_K]
            kf = smf[_S_KF]
            slow_top = jnp.where(k > 0, smf[_S_RELU] + kf * tstar, 0.0)
            neg_top = jnp.where(smi[_I_SLOW] == 1, slow_top,
                                smf[_S_NEG_LOSS])
            pos_i = smf[_S_POS_CNT].astype(jnp.int32)
            denom = (pos_i + k).astype(jnp.float32) + _EPS
            out_ref[0] = (smf[_S_POS_LOSS] + neg_top) / denom


def _balance_bce(pred2d, gt2d, m2d):
    return pl.pallas_call(
        _bce_body,
        grid=(_NROUNDS + 2, _NT),
        in_specs=[
            pl.BlockSpec((_TILE, _COLS),
                         lambda r, t: (jnp.where(r == 0, t, 0), 0)),
            pl.BlockSpec((_TILE, _COLS),
                         lambda r, t: (jnp.where(r == 0, t, 0), 0)),
            pl.BlockSpec((_TILE, _COLS),
                         lambda r, t: (jnp.where(r == 0, t, 0), 0)),
        ],
        out_specs=pl.BlockSpec(memory_space=pltpu.SMEM),
        out_shape=jax.ShapeDtypeStruct((1,), jnp.float32),
        scratch_shapes=[
            pltpu.VMEM((_ROWS, _COLS), jnp.float32),
            pltpu.SMEM((9,), jnp.float32),
            pltpu.SMEM((3,), jnp.int32),
        ],
        compiler_params=pltpu.CompilerParams(
            dimension_semantics=("arbitrary", "arbitrary")),
    )(pred2d, gt2d, m2d)


def kernel(preds, downsample_ratio, gt_shrink, gt_shrink_mask):
    pred2d = preds.reshape(_ROWS, _COLS)
    gt2d = gt_shrink.astype(jnp.int8).reshape(_ROWS, _COLS)
    m2d = gt_shrink_mask.astype(jnp.int8).reshape(_ROWS, _COLS)
    out = _balance_bce(pred2d, gt2d, m2d)
    return out[0] * jnp.float32(1.0) * downsample_ratio
